# Initial kernel scaffold; baseline (speedup 1.0000x reference)
import jax
import jax.numpy as jnp
from jax.experimental import pallas as pl


def kernel(x, edge_index, W, a_src, a_dst):
    raise NotImplementedError("write your pallas kernel here")


# speedup vs baseline: 62.2849x; 62.2849x over previous
import functools

import jax
import jax.numpy as jnp
from jax import lax
from jax.experimental import pallas as pl
from jax.experimental.pallas import tpu as pltpu
from jax.experimental.pallas import tpu_sc as plsc

V = 10000
E = 320000
FIN = 128
H = 4
COUT = 32
HC = H * COUT
NEG_SLOPE = 0.2

NC = 2
NS = 16
NW = NC * NS
EPT = E // NW
EB = 80
NBLK = EPT // EB
NG = EB // 16
VP = 10240
RPT = VP // NS

_ROWS = 1000


def _node_mm_body(x_ref, w_ref, a16_ref, z_ref, e16_ref):
    z = jnp.dot(x_ref[...], w_ref[...], preferred_element_type=jnp.float32)
    z_ref[...] = z
    e16_ref[...] = jnp.dot(z, a16_ref[...], preferred_element_type=jnp.float32)


def _node_mm(x, w, a16):
    return pl.pallas_call(
        _node_mm_body,
        grid=(V // _ROWS,),
        in_specs=[
            pl.BlockSpec((_ROWS, FIN), lambda i: (i, 0)),
            pl.BlockSpec((FIN, HC), lambda i: (0, 0)),
            pl.BlockSpec((FIN, 16), lambda i: (0, 0)),
        ],
        out_specs=[
            pl.BlockSpec((_ROWS, HC), lambda i: (i, 0)),
            pl.BlockSpec((_ROWS, 16), lambda i: (i, 0)),
        ],
        out_shape=[
            jax.ShapeDtypeStruct((V, HC), jnp.float32),
            jax.ShapeDtypeStruct((V, 16), jnp.float32),
        ],
    )(x, w, a16)


def _sc_edge_body(z_hbm, e16_hbm, src_hbm, dst_hbm, outz_hbm, outw_hbm,
                  srcv, dstv, zblk, esblk, edblk, wblk, aggz, aggw, sem):
    c = lax.axis_index("c")
    s = lax.axis_index("s")
    wid = c * NS + s

    def _vgather(vec, idx):
        return lax.gather(
            vec, idx[:, None],
            lax.GatherDimensionNumbers(
                offset_dims=(), collapsed_slice_dims=(0,),
                start_index_map=(0,)),
            (1,), mode=lax.GatherScatterMode.PROMISE_IN_BOUNDS)

    def _zero_row(i, _):
        for j in range(FIN // 16):
            zblk[i, pl.ds(j * 16, 16)] = jnp.zeros((16,), jnp.float32)
        wblk[i, pl.ds(0, 16)] = jnp.zeros((16,), jnp.float32)
        return 0

    lax.fori_loop(0, EB, _zero_row, 0)

    rbase = s * RPT
    for j in range(RPT // EB):
        pltpu.sync_copy(zblk, aggz.at[pl.ds(rbase + j * EB, EB)])
        pltpu.sync_copy(wblk, aggw.at[pl.ds(rbase + j * EB, EB)])
    plsc.subcore_barrier()

    def _block(k, _):
        base = wid * EPT + k * EB
        pltpu.sync_copy(src_hbm.at[pl.ds(base, EB)], srcv)
        pltpu.sync_copy(dst_hbm.at[pl.ds(base, EB)], dstv)
        cz = pltpu.async_copy(z_hbm.at[srcv], zblk, sem)
        cs = pltpu.async_copy(e16_hbm.at[srcv], esblk, sem)
        cd = pltpu.async_copy(e16_hbm.at[dstv], edblk, sem)
        cz.wait()
        cs.wait()
        cd.wait()

        def _edge(e, _):
            lanes = lax.iota(jnp.int32, 16)
            shift8 = jnp.where(lanes < 8, lanes + 8, lanes)
            es = esblk[e, pl.ds(0, 16)]
            ed = _vgather(edblk[e, pl.ds(0, 16)], shift8)
            t = es + ed
            t = jnp.maximum(t, t * NEG_SLOPE)
            w = jnp.where(lanes < H, jnp.exp(t), 0.0)
            wblk[e, pl.ds(0, 16)] = w
            for h in range(H):
                b = _vgather(w, jnp.full((16,), h, jnp.int32))
                for j in range(COUT // 16):
                    sl = pl.ds(h * COUT + j * 16, 16)
                    zblk[e, sl] = zblk[e, sl] * b
            return 0

        lax.fori_loop(0, EB, _edge, 0)
        pltpu.sync_copy(zblk, aggz.at[dstv], add=True)
        pltpu.sync_copy(wblk, aggw.at[dstv], add=True)
        return 0

    lax.fori_loop(0, NBLK, _block, 0)
    plsc.subcore_barrier()

    pltpu.sync_copy(aggz.at[pl.ds(rbase, RPT)],
                    outz_hbm.at[c, pl.ds(rbase, RPT)])
    pltpu.sync_copy(aggw.at[pl.ds(rbase, RPT)],
                    outw_hbm.at[c, pl.ds(rbase, RPT)])


def _sc_edge(z, e16, src, dst):
    mesh = plsc.VectorSubcoreMesh(core_axis_name="c", subcore_axis_name="s")
    f = pl.kernel(
        _sc_edge_body,
        out_type=(
            jax.ShapeDtypeStruct((NC, VP, HC), jnp.float32),
            jax.ShapeDtypeStruct((NC, VP, 16), jnp.float32),
        ),
        mesh=mesh,
        compiler_params=pltpu.CompilerParams(use_tc_tiling_on_sc=False),
        scratch_types=[
            pltpu.VMEM((EB,), jnp.int32),
            pltpu.VMEM((EB,), jnp.int32),
            pltpu.VMEM((EB, FIN), jnp.float32),
            pltpu.VMEM((EB, 16), jnp.float32),
            pltpu.VMEM((EB, 16), jnp.float32),
            pltpu.VMEM((EB, 16), jnp.float32),
            pltpu.VMEM_SHARED((VP, HC), jnp.float32),
            pltpu.VMEM_SHARED((VP, 16), jnp.float32),
            pltpu.SemaphoreType.DMA,
        ],
    )
    return f(z, e16, src, dst)


def _finalize_body(zr, wr, br, out_ref):
    num = zr[0] + zr[1]
    den = wr[0] + wr[1]
    den_b = jnp.dot(den, br[...],
                    preferred_element_type=jnp.float32)
    r = num / (den_b + 1e-9)
    out_ref[...] = jnp.where(r > 0, r, jnp.exp(jnp.minimum(r, 0.0)) - 1.0)


def _finalize(outz, outw, brep):
    return pl.pallas_call(
        _finalize_body,
        grid=(V // _ROWS,),
        in_specs=[
            pl.BlockSpec((NC, _ROWS, HC), lambda i: (0, i, 0)),
            pl.BlockSpec((NC, _ROWS, 16), lambda i: (0, i, 0)),
            pl.BlockSpec((16, HC), lambda i: (0, 0)),
        ],
        out_specs=pl.BlockSpec((_ROWS, HC), lambda i: (i, 0)),
        out_shape=jax.ShapeDtypeStruct((V, HC), jnp.float32),
    )(outz, outw, brep)


def kernel(x, edge_index, W, a_src, a_dst):
    eye = jnp.eye(H, dtype=jnp.float32)
    a_blk_src = (a_src[:, :, None] * eye[:, None, :]).reshape(HC, H)
    a_blk_dst = (a_dst[:, :, None] * eye[:, None, :]).reshape(HC, H)
    zpad = jnp.zeros((HC, H), jnp.float32)
    a16 = jnp.concatenate([a_blk_src, zpad, a_blk_dst, zpad], axis=1)
    z, e16 = _node_mm(x, W, a16)
    src = edge_index[0]
    dst = edge_index[1]
    outz, outw = _sc_edge(z, e16, src, dst)
    brep = jnp.concatenate(
        [jnp.kron(eye, jnp.ones((1, COUT), jnp.float32)),
         jnp.zeros((12, HC), jnp.float32)], axis=0)
    return _finalize(outz, outw, brep)


# speedup vs baseline: 125.2991x; 2.0117x over previous
import functools

import jax
import jax.numpy as jnp
from jax import lax
from jax.experimental import pallas as pl
from jax.experimental.pallas import tpu as pltpu
from jax.experimental.pallas import tpu_sc as plsc

V = 10000
E = 320000
FIN = 128
H = 4
COUT = 32
HC = H * COUT
NEG_SLOPE = 0.2

NC = 2
NS = 16
NW = NC * NS
EPT = E // NW
EB = 80
NBLK = EPT // EB
NG = EB // 16
VP = 10240
RPT = VP // NS

_ROWS = 1000


def _node_mm_body(x_ref, w_ref, a16_ref, z_ref, e16_ref):
    z = jnp.dot(x_ref[...], w_ref[...], preferred_element_type=jnp.float32)
    z_ref[...] = z
    e16_ref[...] = jnp.dot(z, a16_ref[...], preferred_element_type=jnp.float32)


def _node_mm(x, w, a16):
    return pl.pallas_call(
        _node_mm_body,
        grid=(V // _ROWS,),
        in_specs=[
            pl.BlockSpec((_ROWS, FIN), lambda i: (i, 0)),
            pl.BlockSpec((FIN, HC), lambda i: (0, 0)),
            pl.BlockSpec((FIN, 16), lambda i: (0, 0)),
        ],
        out_specs=[
            pl.BlockSpec((_ROWS, HC), lambda i: (i, 0)),
            pl.BlockSpec((_ROWS, 16), lambda i: (i, 0)),
        ],
        out_shape=[
            jax.ShapeDtypeStruct((V, HC), jnp.float32),
            jax.ShapeDtypeStruct((V, 16), jnp.float32),
        ],
    )(x, w, a16)


def _sc_edge_body(z_hbm, e16_hbm, src_hbm, dst_hbm, outz_hbm, outw_hbm,
                  srcv0, dstv0, zblk0, esblk0, edblk0, wblk0,
                  srcv1, dstv1, zblk1, esblk1, edblk1, wblk1,
                  aggz, aggw, sem0, sem1):
    c = lax.axis_index("c")
    s = lax.axis_index("s")
    wid = c * NS + s
    bufA = (srcv0, dstv0, zblk0, esblk0, edblk0, wblk0, sem0)
    bufB = (srcv1, dstv1, zblk1, esblk1, edblk1, wblk1, sem1)

    def _vgather(vec, idx):
        return lax.gather(
            vec, idx[:, None],
            lax.GatherDimensionNumbers(
                offset_dims=(), collapsed_slice_dims=(0,),
                start_index_map=(0,)),
            (1,), mode=lax.GatherScatterMode.PROMISE_IN_BOUNDS)

    def _zero_row(i, _):
        for j in range(FIN // 16):
            zblk0[i, pl.ds(j * 16, 16)] = jnp.zeros((16,), jnp.float32)
        wblk0[i, pl.ds(0, 16)] = jnp.zeros((16,), jnp.float32)
        return 0

    lax.fori_loop(0, EB, _zero_row, 0)

    rbase = s * RPT
    for j in range(RPT // EB):
        pltpu.sync_copy(zblk0, aggz.at[pl.ds(rbase + j * EB, EB)])
        pltpu.sync_copy(wblk0, aggw.at[pl.ds(rbase + j * EB, EB)])
    plsc.subcore_barrier()

    def _issue(k, buf):
        srcv, dstv, zblk, esblk, edblk, _, sem = buf
        base = wid * EPT + k * EB
        pltpu.sync_copy(src_hbm.at[pl.ds(base, EB)], srcv)
        pltpu.sync_copy(dst_hbm.at[pl.ds(base, EB)], dstv)
        pltpu.async_copy(z_hbm.at[srcv], zblk, sem)
        pltpu.async_copy(e16_hbm.at[srcv], esblk, sem)
        pltpu.async_copy(e16_hbm.at[dstv], edblk, sem)

    def _drain(buf):
        srcv, dstv, zblk, esblk, edblk, _, sem = buf
        pltpu.make_async_copy(z_hbm.at[srcv], zblk, sem).wait()
        pltpu.make_async_copy(e16_hbm.at[srcv], esblk, sem).wait()
        pltpu.make_async_copy(e16_hbm.at[dstv], edblk, sem).wait()

    def _process(buf):
        srcv, dstv, zblk, esblk, edblk, wblk, sem = buf

        @plsc.parallel_loop(0, EB, step=1, unroll=2)
        def _edge(e):
            lanes = lax.iota(jnp.int32, 16)
            shift8 = jnp.where(lanes < 8, lanes + 8, lanes)
            es = esblk[e, pl.ds(0, 16)]
            ed = _vgather(edblk[e, pl.ds(0, 16)], shift8)
            t = es + ed
            t = jnp.maximum(t, t * NEG_SLOPE)
            w = jnp.where(lanes < H, jnp.exp(t), 0.0)
            wblk[e, pl.ds(0, 16)] = w
            for h in range(H):
                b = _vgather(w, jnp.full((16,), h, jnp.int32))
                for j in range(COUT // 16):
                    sl = pl.ds(h * COUT + j * 16, 16)
                    zblk[e, sl] = zblk[e, sl] * b

        pltpu.sync_copy(zblk, aggz.at[dstv], add=True)
        pltpu.sync_copy(wblk, aggw.at[dstv], add=True)

    _issue(0, bufA)

    def _pair(p, _):
        _issue(2 * p + 1, bufB)
        _drain(bufA)
        _process(bufA)
        _issue(2 * p + 2, bufA)
        _drain(bufB)
        _process(bufB)
        return 0

    lax.fori_loop(0, (NBLK - 1) // 2, _pair, 0)
    _drain(bufA)
    _process(bufA)
    plsc.subcore_barrier()

    pltpu.sync_copy(aggz.at[pl.ds(rbase, RPT)],
                    outz_hbm.at[c, pl.ds(rbase, RPT)])
    pltpu.sync_copy(aggw.at[pl.ds(rbase, RPT)],
                    outw_hbm.at[c, pl.ds(rbase, RPT)])


def _sc_edge(z, e16, src, dst):
    mesh = plsc.VectorSubcoreMesh(core_axis_name="c", subcore_axis_name="s")
    f = pl.kernel(
        _sc_edge_body,
        out_type=(
            jax.ShapeDtypeStruct((NC, VP, HC), jnp.float32),
            jax.ShapeDtypeStruct((NC, VP, 16), jnp.float32),
        ),
        mesh=mesh,
        compiler_params=pltpu.CompilerParams(use_tc_tiling_on_sc=False),
        scratch_types=[
            pltpu.VMEM((EB,), jnp.int32),
            pltpu.VMEM((EB,), jnp.int32),
            pltpu.VMEM((EB, FIN), jnp.float32),
            pltpu.VMEM((EB, 16), jnp.float32),
            pltpu.VMEM((EB, 16), jnp.float32),
            pltpu.VMEM((EB, 16), jnp.float32),
            pltpu.VMEM((EB,), jnp.int32),
            pltpu.VMEM((EB,), jnp.int32),
            pltpu.VMEM((EB, FIN), jnp.float32),
            pltpu.VMEM((EB, 16), jnp.float32),
            pltpu.VMEM((EB, 16), jnp.float32),
            pltpu.VMEM((EB, 16), jnp.float32),
            pltpu.VMEM_SHARED((VP, HC), jnp.float32),
            pltpu.VMEM_SHARED((VP, 16), jnp.float32),
            pltpu.SemaphoreType.DMA,
            pltpu.SemaphoreType.DMA,
        ],
    )
    return f(z, e16, src, dst)


def _finalize_body(zr, wr, br, out_ref):
    num = zr[0] + zr[1]
    den = wr[0] + wr[1]
    den_b = jnp.dot(den, br[...],
                    preferred_element_type=jnp.float32)
    r = num / (den_b + 1e-9)
    out_ref[...] = jnp.where(r > 0, r, jnp.exp(jnp.minimum(r, 0.0)) - 1.0)


def _finalize(outz, outw, brep):
    return pl.pallas_call(
        _finalize_body,
        grid=(V // _ROWS,),
        in_specs=[
            pl.BlockSpec((NC, _ROWS, HC), lambda i: (0, i, 0)),
            pl.BlockSpec((NC, _ROWS, 16), lambda i: (0, i, 0)),
            pl.BlockSpec((16, HC), lambda i: (0, 0)),
        ],
        out_specs=pl.BlockSpec((_ROWS, HC), lambda i: (i, 0)),
        out_shape=jax.ShapeDtypeStruct((V, HC), jnp.float32),
    )(outz, outw, brep)


def kernel(x, edge_index, W, a_src, a_dst):
    eye = jnp.eye(H, dtype=jnp.float32)
    a_blk_src = (a_src[:, :, None] * eye[:, None, :]).reshape(HC, H)
    a_blk_dst = (a_dst[:, :, None] * eye[:, None, :]).reshape(HC, H)
    zpad = jnp.zeros((HC, H), jnp.float32)
    a16 = jnp.concatenate([a_blk_src, zpad, a_blk_dst, zpad], axis=1)
    z, e16 = _node_mm(x, W, a16)
    src = edge_index[0]
    dst = edge_index[1]
    outz, outw = _sc_edge(z, e16, src, dst)
    brep = jnp.concatenate(
        [jnp.kron(eye, jnp.ones((1, COUT), jnp.float32)),
         jnp.zeros((12, HC), jnp.float32)], axis=0)
    return _finalize(outz, outw, brep)


# speedup vs baseline: 125.4496x; 1.0012x over previous
import functools

import jax
import jax.numpy as jnp
from jax import lax
from jax.experimental import pallas as pl
from jax.experimental.pallas import tpu as pltpu
from jax.experimental.pallas import tpu_sc as plsc

V = 10000
E = 320000
FIN = 128
H = 4
COUT = 32
HC = H * COUT
NEG_SLOPE = 0.2

NC = 2
NS = 16
NW = NC * NS
EPT = E // NW
EB = 80
NBLK = EPT // EB
NG = EB // 16
VP = 10240
RPT = VP // NS

_ROWS = 1000


def _node_mm_body(x_ref, w_ref, a16_ref, z_ref, e16_ref):
    z = jnp.dot(x_ref[...], w_ref[...], preferred_element_type=jnp.float32)
    z_ref[...] = z
    e16_ref[...] = jnp.dot(z, a16_ref[...], preferred_element_type=jnp.float32)


def _node_mm(x, w, a16):
    return pl.pallas_call(
        _node_mm_body,
        grid=(V // _ROWS,),
        in_specs=[
            pl.BlockSpec((_ROWS, FIN), lambda i: (i, 0)),
            pl.BlockSpec((FIN, HC), lambda i: (0, 0)),
            pl.BlockSpec((FIN, 16), lambda i: (0, 0)),
        ],
        out_specs=[
            pl.BlockSpec((_ROWS, HC), lambda i: (i, 0)),
            pl.BlockSpec((_ROWS, 16), lambda i: (i, 0)),
        ],
        out_shape=[
            jax.ShapeDtypeStruct((V, HC), jnp.float32),
            jax.ShapeDtypeStruct((V, 16), jnp.float32),
        ],
    )(x, w, a16)


def _sc_edge_body(z_hbm, e16_hbm, src_hbm, dst_hbm, outz_hbm, outw_hbm,
                  srcv0, dstv0, zblk0, esblk0, edblk0, wblk0,
                  srcv1, dstv1, zblk1, esblk1, edblk1, wblk1,
                  aggz, aggw, sem0, sem1):
    c = lax.axis_index("c")
    s = lax.axis_index("s")
    wid = c * NS + s
    bufA = (srcv0, dstv0, zblk0, esblk0, edblk0, wblk0, sem0)
    bufB = (srcv1, dstv1, zblk1, esblk1, edblk1, wblk1, sem1)

    def _vgather(vec, idx):
        return lax.gather(
            vec, idx[:, None],
            lax.GatherDimensionNumbers(
                offset_dims=(), collapsed_slice_dims=(0,),
                start_index_map=(0,)),
            (1,), mode=lax.GatherScatterMode.PROMISE_IN_BOUNDS)

    def _zero_row(i, _):
        for j in range(FIN // 16):
            zblk0[i, pl.ds(j * 16, 16)] = jnp.zeros((16,), jnp.float32)
        wblk0[i, pl.ds(0, 16)] = jnp.zeros((16,), jnp.float32)
        return 0

    lax.fori_loop(0, EB, _zero_row, 0)

    rbase = s * RPT
    for j in range(RPT // EB):
        pltpu.sync_copy(zblk0, aggz.at[pl.ds(rbase + j * EB, EB)])
        pltpu.sync_copy(wblk0, aggw.at[pl.ds(rbase + j * EB, EB)])
    plsc.subcore_barrier()

    def _issue(k, buf):
        srcv, dstv, zblk, esblk, edblk, _, sem = buf
        base = wid * EPT + k * EB
        pltpu.sync_copy(src_hbm.at[pl.ds(base, EB)], srcv)
        pltpu.sync_copy(dst_hbm.at[pl.ds(base, EB)], dstv)
        pltpu.async_copy(z_hbm.at[srcv], zblk, sem)
        pltpu.async_copy(e16_hbm.at[srcv], esblk, sem)
        pltpu.async_copy(e16_hbm.at[dstv], edblk, sem)

    def _drain(buf):
        srcv, dstv, zblk, esblk, edblk, _, sem = buf
        pltpu.make_async_copy(z_hbm.at[srcv], zblk, sem).wait()
        pltpu.make_async_copy(e16_hbm.at[srcv], esblk, sem).wait()
        pltpu.make_async_copy(e16_hbm.at[dstv], edblk, sem).wait()

    def _process(buf):
        srcv, dstv, zblk, esblk, edblk, wblk, sem = buf

        @plsc.parallel_loop(0, EB, step=1, unroll=4)
        def _edge(e):
            lanes = lax.iota(jnp.int32, 16)
            shift8 = jnp.where(lanes < 8, lanes + 8, lanes)
            es = esblk[e, pl.ds(0, 16)]
            ed = _vgather(edblk[e, pl.ds(0, 16)], shift8)
            t = es + ed
            t = jnp.maximum(t, t * NEG_SLOPE)
            w = jnp.where(lanes < H, jnp.exp(t), 0.0)
            wblk[e, pl.ds(0, 16)] = w
            for h in range(H):
                b = _vgather(w, jnp.full((16,), h, jnp.int32))
                for j in range(COUT // 16):
                    sl = pl.ds(h * COUT + j * 16, 16)
                    zblk[e, sl] = zblk[e, sl] * b

        pltpu.sync_copy(zblk, aggz.at[dstv], add=True)
        pltpu.sync_copy(wblk, aggw.at[dstv], add=True)

    _issue(0, bufA)

    def _pair(p, _):
        _issue(2 * p + 1, bufB)
        _drain(bufA)
        _process(bufA)
        _issue(2 * p + 2, bufA)
        _drain(bufB)
        _process(bufB)
        return 0

    lax.fori_loop(0, (NBLK - 1) // 2, _pair, 0)
    _drain(bufA)
    _process(bufA)
    plsc.subcore_barrier()

    pltpu.sync_copy(aggz.at[pl.ds(rbase, RPT)],
                    outz_hbm.at[c, pl.ds(rbase, RPT)])
    pltpu.sync_copy(aggw.at[pl.ds(rbase, RPT)],
                    outw_hbm.at[c, pl.ds(rbase, RPT)])


def _sc_edge(z, e16, src, dst):
    mesh = plsc.VectorSubcoreMesh(core_axis_name="c", subcore_axis_name="s")
    f = pl.kernel(
        _sc_edge_body,
        out_type=(
            jax.ShapeDtypeStruct((NC, VP, HC), jnp.float32),
            jax.ShapeDtypeStruct((NC, VP, 16), jnp.float32),
        ),
        mesh=mesh,
        compiler_params=pltpu.CompilerParams(use_tc_tiling_on_sc=False),
        scratch_types=[
            pltpu.VMEM((EB,), jnp.int32),
            pltpu.VMEM((EB,), jnp.int32),
            pltpu.VMEM((EB, FIN), jnp.float32),
            pltpu.VMEM((EB, 16), jnp.float32),
            pltpu.VMEM((EB, 16), jnp.float32),
            pltpu.VMEM((EB, 16), jnp.float32),
            pltpu.VMEM((EB,), jnp.int32),
            pltpu.VMEM((EB,), jnp.int32),
            pltpu.VMEM((EB, FIN), jnp.float32),
            pltpu.VMEM((EB, 16), jnp.float32),
            pltpu.VMEM((EB, 16), jnp.float32),
            pltpu.VMEM((EB, 16), jnp.float32),
            pltpu.VMEM_SHARED((VP, HC), jnp.float32),
            pltpu.VMEM_SHARED((VP, 16), jnp.float32),
            pltpu.SemaphoreType.DMA,
            pltpu.SemaphoreType.DMA,
        ],
    )
    return f(z, e16, src, dst)


def _finalize_body(zr, wr, br, out_ref):
    num = zr[0] + zr[1]
    den = wr[0] + wr[1]
    den_b = jnp.dot(den, br[...],
                    preferred_element_type=jnp.float32)
    r = num / (den_b + 1e-9)
    out_ref[...] = jnp.where(r > 0, r, jnp.exp(jnp.minimum(r, 0.0)) - 1.0)


def _finalize(outz, outw, brep):
    return pl.pallas_call(
        _finalize_body,
        grid=(V // _ROWS,),
        in_specs=[
            pl.BlockSpec((NC, _ROWS, HC), lambda i: (0, i, 0)),
            pl.BlockSpec((NC, _ROWS, 16), lambda i: (0, i, 0)),
            pl.BlockSpec((16, HC), lambda i: (0, 0)),
        ],
        out_specs=pl.BlockSpec((_ROWS, HC), lambda i: (i, 0)),
        out_shape=jax.ShapeDtypeStruct((V, HC), jnp.float32),
    )(outz, outw, brep)


def kernel(x, edge_index, W, a_src, a_dst):
    eye = jnp.eye(H, dtype=jnp.float32)
    a_blk_src = (a_src[:, :, None] * eye[:, None, :]).reshape(HC, H)
    a_blk_dst = (a_dst[:, :, None] * eye[:, None, :]).reshape(HC, H)
    zpad = jnp.zeros((HC, H), jnp.float32)
    a16 = jnp.concatenate([a_blk_src, zpad, a_blk_dst, zpad], axis=1)
    z, e16 = _node_mm(x, W, a16)
    src = edge_index[0]
    dst = edge_index[1]
    outz, outw = _sc_edge(z, e16, src, dst)
    brep = jnp.concatenate(
        [jnp.kron(eye, jnp.ones((1, COUT), jnp.float32)),
         jnp.zeros((12, HC), jnp.float32)], axis=0)
    return _finalize(outz, outw, brep)
